# Initial kernel scaffold; baseline (speedup 1.0000x reference)
import jax
import jax.numpy as jnp
from jax.experimental import pallas as pl


def kernel(q_values, v_pred, rewards, actions, weights, ratio, gamma):
    raise NotImplementedError("write your pallas kernel here")


# speedup vs baseline: 2.6287x; 2.6287x over previous
import jax
import jax.numpy as jnp
from jax import lax
from jax.experimental import pallas as pl
from jax.experimental.pallas import tpu as pltpu
from jax.experimental.pallas import tpu_sc as plsc

_T, _B, _N = 512, 1024, 64
_NC, _NS = 2, 16
_NW = _NC * _NS
_CB = _B // _NW
_E = _T * _CB
_CHUNK = 128
_NCH = _E // _CHUNK


def _retrace_body(qv_hbm, rat_hbm, act_hbm, rew_hbm, wts_hbm, vp_hbm, gam_hbm,
                  out_hbm,
                  act_v, idx_v, qg_v, rg_v, rew_v, wts_v, vp_v, gam_v,
                  sem_a, sem):
    wid = lax.axis_index("s") * _NC + lax.axis_index("c")
    base = wid * _CB
    lanes = lax.broadcasted_iota(jnp.int32, (16,), 0)

    pltpu.async_copy(act_hbm.at[:, pl.ds(base, _CB)], act_v, sem_a)
    pltpu.async_copy(rew_hbm.at[:, pl.ds(base, _CB)], rew_v, sem)
    pltpu.async_copy(wts_hbm.at[:, pl.ds(base, _CB)], wts_v, sem)
    pltpu.async_copy(vp_hbm.at[:, pl.ds(base, _CB)], vp_v, sem)
    pltpu.sync_copy(gam_hbm, gam_v)
    pltpu.make_async_copy(act_hbm.at[:, pl.ds(base, _CB)], act_v, sem_a).wait()

    def idx_body(v, c):
        t = v >> 1
        s16 = (v & 1) * 16
        a = act_v[t, pl.ds(s16, 16)]
        fi = t * (_B * _N) + (base + s16 + lanes) * _N + a
        idx_v[v >> 3, pl.ds((v & 7) * 16, 16)] = fi
        return c
    lax.fori_loop(0, _E // 16, idx_body, 0)

    def fire(c, carry):
        pltpu.async_copy(qv_hbm.at[idx_v.at[c]], qg_v.at[c], sem)
        pltpu.async_copy(rat_hbm.at[idx_v.at[c]], rg_v.at[c], sem)
        return carry
    lax.fori_loop(0, _NCH, fire, 0)

    def drain(c, carry):
        pltpu.make_async_copy(qv_hbm.at[idx_v.at[c]], qg_v.at[c], sem).wait()
        pltpu.make_async_copy(rat_hbm.at[idx_v.at[c]], rg_v.at[c], sem).wait()
        return carry
    lax.fori_loop(0, _NCH, drain, 0)
    pltpu.make_async_copy(rew_hbm.at[:, pl.ds(base, _CB)], rew_v, sem).wait()
    pltpu.make_async_copy(wts_hbm.at[:, pl.ds(base, _CB)], wts_v, sem).wait()
    pltpu.make_async_copy(vp_hbm.at[:, pl.ds(base, _CB)], vp_v, sem).wait()

    g = gam_v[...]
    one = jnp.full((16,), 1.0, jnp.float32)

    def scan_body(i, carry):
        t0, t1 = carry
        t = _T - 1 - i
        flat = t * _CB
        row = flat >> 7
        col = flat & 127
        r0 = rew_v[t, pl.ds(0, 16)]
        r1 = rew_v[t, pl.ds(16, 16)]
        w0 = wts_v[t, pl.ds(0, 16)]
        w1 = wts_v[t, pl.ds(16, 16)]
        rg0 = rg_v[row, pl.ds(col, 16)]
        rg1 = rg_v[row, pl.ds(col + 16, 16)]
        qg0 = qg_v[row, pl.ds(col, 16)]
        qg1 = qg_v[row, pl.ds(col + 16, 16)]
        v0 = vp_v[t, pl.ds(0, 16)]
        v1 = vp_v[t, pl.ds(16, 16)]
        qr0 = r0 + g * w0 * t0
        qr1 = r1 + g * w1 * t1
        n0 = jnp.minimum(rg0, one) * (qr0 - qg0) + v0
        n1 = jnp.minimum(rg1, one) * (qr1 - qg1) + v1
        rew_v[t, pl.ds(0, 16)] = qr0
        rew_v[t, pl.ds(16, 16)] = qr1
        return (n0, n1)

    tmp0 = vp_v[_T, pl.ds(0, 16)]
    tmp1 = vp_v[_T, pl.ds(16, 16)]
    lax.fori_loop(0, _T, scan_body, (tmp0, tmp1))

    pltpu.async_copy(rew_v, out_hbm.at[pl.ds(0, _T), pl.ds(base, _CB)], sem_a)
    pltpu.async_copy(vp_v.at[pl.ds(_T, 1)],
                     out_hbm.at[pl.ds(_T, 1), pl.ds(base, _CB)], sem_a)
    pltpu.make_async_copy(rew_v,
                          out_hbm.at[pl.ds(0, _T), pl.ds(base, _CB)],
                          sem_a).wait()
    pltpu.make_async_copy(vp_v.at[pl.ds(_T, 1)],
                          out_hbm.at[pl.ds(_T, 1), pl.ds(base, _CB)],
                          sem_a).wait()


def kernel(q_values, v_pred, rewards, actions, weights, ratio, gamma):
    qv_flat = q_values.reshape(-1)
    rat_flat = ratio.reshape(-1)
    act = actions.astype(jnp.int32)
    vp = v_pred.reshape(_T + 1, _B)
    gam = jnp.full((16,), gamma, jnp.float32)

    run = pl.kernel(
        _retrace_body,
        out_type=jax.ShapeDtypeStruct((_T + 1, _B), jnp.float32),
        mesh=plsc.VectorSubcoreMesh(core_axis_name="c", subcore_axis_name="s"),
        compiler_params=pltpu.CompilerParams(use_tc_tiling_on_sc=False),
        scratch_types=[
            pltpu.VMEM((_T, _CB), jnp.int32),
            pltpu.VMEM((_NCH, _CHUNK), jnp.int32),
            pltpu.VMEM((_NCH, _CHUNK), jnp.float32),
            pltpu.VMEM((_NCH, _CHUNK), jnp.float32),
            pltpu.VMEM((_T, _CB), jnp.float32),
            pltpu.VMEM((_T, _CB), jnp.float32),
            pltpu.VMEM((_T + 1, _CB), jnp.float32),
            pltpu.VMEM((16,), jnp.float32),
            pltpu.SemaphoreType.DMA,
            pltpu.SemaphoreType.DMA,
        ],
    )
    out = run(qv_flat, rat_flat, act, rewards, weights, vp, gam)
    return out.reshape(_T + 1, _B, 1)


# speedup vs baseline: 23.7837x; 9.0478x over previous
import jax
import jax.numpy as jnp
from jax import lax
from jax.experimental import pallas as pl
from jax.experimental.pallas import tpu as pltpu
from jax.experimental.pallas import tpu_sc as plsc

_T, _B, _N = 512, 1024, 64
_NC, _NS = 2, 16
_NW = _NC * _NS
_CB = _B // _NW
_E = _T * _CB
_CHUNK = 128
_NCH = _E // _CHUNK
_TPC = _CHUNK // _CB


def _retrace_body(qv_hbm, rat_hbm, act_hbm, rew_hbm, wts_hbm, vp_hbm, gam_hbm,
                  out_hbm,
                  act_v, idx_v, qg_v, rg_v, rew_v, wts_v, vp_v, gam_v,
                  sem_a, sem, sem_q, sem_r):
    wid = lax.axis_index("s") * _NC + lax.axis_index("c")
    base = wid * _CB
    lanes = lax.broadcasted_iota(jnp.int32, (16,), 0)

    pltpu.async_copy(act_hbm.at[:, pl.ds(base, _CB)], act_v, sem_a)
    pltpu.async_copy(rew_hbm.at[:, pl.ds(base, _CB)], rew_v, sem)
    pltpu.async_copy(wts_hbm.at[:, pl.ds(base, _CB)], wts_v, sem)
    pltpu.async_copy(vp_hbm.at[:, pl.ds(base, _CB)], vp_v, sem)
    pltpu.sync_copy(gam_hbm, gam_v)
    pltpu.make_async_copy(act_hbm.at[:, pl.ds(base, _CB)], act_v, sem_a).wait()

    bq0 = ((base + lanes) >> 7) * 1024 + ((base + lanes) & 127)
    bq1 = ((base + 16 + lanes) >> 7) * 1024 + ((base + 16 + lanes) & 127)

    def fire_body(i, carry):
        c = _NCH - 1 - i
        for j in range(_TPC):
            t = c * _TPC + j
            tb = t * (_B * _N)
            a0 = act_v[t, pl.ds(0, 16)]
            a1 = act_v[t, pl.ds(16, 16)]
            f0 = tb + (a0 >> 3) * (8 * _B) + (a0 & 7) * 128 + bq0
            f1 = tb + (a1 >> 3) * (8 * _B) + (a1 & 7) * 128 + bq1
            idx_v[c, pl.ds(j * _CB, 16)] = f0
            idx_v[c, pl.ds(j * _CB + 16, 16)] = f1
        pltpu.async_copy(qv_hbm.at[idx_v.at[c]], qg_v.at[c], sem_q)
        pltpu.async_copy(rat_hbm.at[idx_v.at[c]], rg_v.at[c], sem_r)
        return carry
    lax.fori_loop(0, _NCH, fire_body, 0)

    pltpu.make_async_copy(rew_hbm.at[:, pl.ds(base, _CB)], rew_v, sem).wait()
    pltpu.make_async_copy(wts_hbm.at[:, pl.ds(base, _CB)], wts_v, sem).wait()
    pltpu.make_async_copy(vp_hbm.at[:, pl.ds(base, _CB)], vp_v, sem).wait()

    g = gam_v[...]
    one = jnp.full((16,), 1.0, jnp.float32)

    def scan_chunk(i, carry):
        c = _NCH - 1 - i
        pltpu.make_async_copy(qv_hbm.at[idx_v.at[c]], qg_v.at[c], sem_q).wait()
        pltpu.make_async_copy(rat_hbm.at[idx_v.at[c]], rg_v.at[c], sem_r).wait()
        t0, t1 = carry
        for j in reversed(range(_TPC)):
            t = c * _TPC + j
            col = j * _CB
            r0 = rew_v[t, pl.ds(0, 16)]
            r1 = rew_v[t, pl.ds(16, 16)]
            w0 = wts_v[t, pl.ds(0, 16)]
            w1 = wts_v[t, pl.ds(16, 16)]
            rg0 = rg_v[c, pl.ds(col, 16)]
            rg1 = rg_v[c, pl.ds(col + 16, 16)]
            qg0 = qg_v[c, pl.ds(col, 16)]
            qg1 = qg_v[c, pl.ds(col + 16, 16)]
            v0 = vp_v[t, pl.ds(0, 16)]
            v1 = vp_v[t, pl.ds(16, 16)]
            qr0 = r0 + g * w0 * t0
            qr1 = r1 + g * w1 * t1
            t0 = jnp.minimum(rg0, one) * (qr0 - qg0) + v0
            t1 = jnp.minimum(rg1, one) * (qr1 - qg1) + v1
            rew_v[t, pl.ds(0, 16)] = qr0
            rew_v[t, pl.ds(16, 16)] = qr1
        return (t0, t1)

    tmp0 = vp_v[_T, pl.ds(0, 16)]
    tmp1 = vp_v[_T, pl.ds(16, 16)]
    lax.fori_loop(0, _NCH, scan_chunk, (tmp0, tmp1))

    pltpu.async_copy(rew_v, out_hbm.at[pl.ds(0, _T), pl.ds(base, _CB)], sem_a)
    pltpu.async_copy(vp_v.at[pl.ds(_T, 1)],
                     out_hbm.at[pl.ds(_T, 1), pl.ds(base, _CB)], sem_a)
    pltpu.make_async_copy(rew_v,
                          out_hbm.at[pl.ds(0, _T), pl.ds(base, _CB)],
                          sem_a).wait()
    pltpu.make_async_copy(vp_v.at[pl.ds(_T, 1)],
                          out_hbm.at[pl.ds(_T, 1), pl.ds(base, _CB)],
                          sem_a).wait()


def _tile_order_flat(x):
    tp = x.shape[0]
    xt = jnp.transpose(x, (0, 2, 1))
    x5 = xt.reshape(tp, _N // 8, 8, _B // 128, 128)
    x5 = jnp.transpose(x5, (0, 1, 3, 2, 4))
    return x5.reshape(-1)


def kernel(q_values, v_pred, rewards, actions, weights, ratio, gamma):
    qv_flat = _tile_order_flat(q_values)
    rat_flat = _tile_order_flat(ratio)
    act = actions.astype(jnp.int32)
    vp = v_pred.reshape(_T + 1, _B)
    gam = jnp.full((16,), gamma, jnp.float32)

    run = pl.kernel(
        _retrace_body,
        out_type=jax.ShapeDtypeStruct((_T + 1, _B), jnp.float32),
        mesh=plsc.VectorSubcoreMesh(core_axis_name="c", subcore_axis_name="s"),
        compiler_params=pltpu.CompilerParams(use_tc_tiling_on_sc=False),
        scratch_types=[
            pltpu.VMEM((_T, _CB), jnp.int32),
            pltpu.VMEM((_NCH, _CHUNK), jnp.int32),
            pltpu.VMEM((_NCH, _CHUNK), jnp.float32),
            pltpu.VMEM((_NCH, _CHUNK), jnp.float32),
            pltpu.VMEM((_T, _CB), jnp.float32),
            pltpu.VMEM((_T, _CB), jnp.float32),
            pltpu.VMEM((_T + 1, _CB), jnp.float32),
            pltpu.VMEM((16,), jnp.float32),
            pltpu.SemaphoreType.DMA,
            pltpu.SemaphoreType.DMA,
            pltpu.SemaphoreType.DMA,
            pltpu.SemaphoreType.DMA,
        ],
    )
    out = run(qv_flat, rat_flat, act, rewards, weights, vp, gam)
    return out.reshape(_T + 1, _B, 1)
